# Initial kernel scaffold; baseline (speedup 1.0000x reference)
import jax
import jax.numpy as jnp
from jax.experimental import pallas as pl


def kernel(ref_rgb_feat, ref_flow_feat, sup_rgb_feat, sup_flow_feat):
    raise NotImplementedError("write your pallas kernel here")


# speedup vs baseline: 1.5675x; 1.5675x over previous
import functools
import jax
import jax.numpy as jnp
from jax import lax
from jax.experimental import pallas as pl
from jax.experimental.pallas import tpu as pltpu
from jax.experimental.pallas import tpu_sc as plsc

L = 16


def _round_bf16(x):
    return x.astype(jnp.bfloat16).astype(jnp.float32)


def _tc_stage1_body(flow_ref, s_ref):
    n = flow_ref.shape[1]
    xr = _round_bf16(flow_ref[...])
    mu = jnp.sum(xr, axis=1, keepdims=True) / jnp.float32(n)
    s_ref[...] = jnp.sum(xr * mu, axis=2)[:, None, :]


def _tc_stage2_body(sup_ref, km_ref, s_ref):
    xr = _round_bf16(sup_ref[...])
    km = km_ref[...]
    s_ref[...] = jnp.sum(xr * km, axis=2)[:, None, :]


def _stage1_scores(flow):
    b, n, f = flow.shape
    s = pl.pallas_call(
        _tc_stage1_body,
        grid=(b,),
        in_specs=[pl.BlockSpec((1, n, f), lambda i: (i, 0, 0))],
        out_specs=pl.BlockSpec((1, 1, n), lambda i: (i, 0, 0)),
        out_shape=jax.ShapeDtypeStruct((b, 1, n), jnp.float32),
    )(flow)
    return s.reshape(b, n)


def _stage2_scores(sup_flow, kmean):
    b, m, f = sup_flow.shape
    s = pl.pallas_call(
        _tc_stage2_body,
        grid=(b,),
        in_specs=[
            pl.BlockSpec((1, m, f), lambda i: (i, 0, 0)),
            pl.BlockSpec((1, 1, f), lambda i: (i, 0, 0)),
        ],
        out_specs=pl.BlockSpec((1, 1, m), lambda i: (i, 0, 0)),
        out_shape=jax.ShapeDtypeStruct((b, 1, m), jnp.float32),
    )(sup_flow, kmean.reshape(b, 1, f))
    return s.reshape(b, m)


def _sc_round_bf16(x):
    w = plsc.bitcast(x, jnp.int32)
    tie = lax.shift_right_logical(w, 16) & jnp.int32(1)
    wr = (w + jnp.int32(0x7FFF) + tie) & jnp.int32(-65536)
    return plsc.bitcast(wr, jnp.float32)


def _lane_iota():
    return lax.iota(jnp.int32, 16)


def _scan_topk(s_v, n, k, largest):
    groups = n // L
    sentinel = jnp.float32(-jnp.inf) if largest else jnp.float32(jnp.inf)
    iota = _lane_iota()

    def better(a, bv):
        return a > bv if largest else a < bv

    def body(g, carry):
        m1, i1, m2, i2, m3, i3 = carry
        s = s_v[pl.ds(g * L, L)]
        nvec = g * L + iota
        c1 = better(s, m1)
        c2 = better(s, m2)
        c3 = better(s, m3)
        nm3 = jnp.where(c2, m2, jnp.where(c3, s, m3))
        ni3 = jnp.where(c2, i2, jnp.where(c3, nvec, i3))
        nm2 = jnp.where(c1, m1, jnp.where(c2, s, m2))
        ni2 = jnp.where(c1, i1, jnp.where(c2, nvec, i2))
        nm1 = jnp.where(c1, s, m1)
        ni1 = jnp.where(c1, nvec, i1)
        return nm1, ni1, nm2, ni2, nm3, ni3

    full = jnp.full((L,), sentinel, jnp.float32)
    zero = jnp.zeros((L,), jnp.int32)
    m1, i1, m2, i2, m3, i3 = lax.fori_loop(
        0, groups, body, (full, zero, full, zero, full, zero))

    idx_g = jnp.zeros((L,), jnp.int32)
    big = jnp.int32(2 ** 30)
    for r in range(k):
        gbest = jnp.max(m1) if largest else jnp.min(m1)
        isel = jnp.min(jnp.where(m1 == gbest, i1, big))
        upd = (m1 == gbest) & (i1 == isel)
        idx_g = jnp.where(iota == r, isel, idx_g)
        m1 = jnp.where(upd, m2, m1)
        i1 = jnp.where(upd, i2, i1)
        m2 = jnp.where(upd, m3, m2)
        i2 = jnp.where(upd, i3, i2)
        m3 = jnp.where(upd, jnp.full((L,), sentinel, jnp.float32), m3)
    return idx_g


def _make_sc1(b, n, f):
    mesh = plsc.VectorSubcoreMesh(core_axis_name="c", subcore_axis_name="s")

    @functools.partial(
        pl.kernel,
        mesh=mesh,
        out_type=(
            jax.ShapeDtypeStruct((b, 3, f), jnp.float32),
            jax.ShapeDtypeStruct((b, f), jnp.float32),
        ),
        compiler_params=pltpu.CompilerParams(needs_layout_passes=False, use_tc_tiling_on_sc=False),
        scratch_types=[
            pltpu.VMEM((n,), jnp.float32),
            pltpu.VMEM((L, f), jnp.float32),
            pltpu.VMEM((L, f), jnp.float32),
            pltpu.VMEM((2 * L,), jnp.float32),
            pltpu.SemaphoreType.DMA,
            pltpu.SemaphoreType.DMA,
        ],
    )
    def sc1(s_hbm, rgb_hbm, flow_hbm, krgb_hbm, km_hbm, s_v, rows_rgb,
            rows_flow, km_v, sem1, sem2):
        cid = lax.axis_index("c")
        sid = lax.axis_index("s")

        @pl.when(cid == 0)
        def _():
            batch = sid
            pltpu.sync_copy(s_hbm.at[batch], s_v)
            idx_g = _scan_topk(s_v, n, 3, largest=False)
            cp1 = pltpu.async_copy(rgb_hbm.at[batch].at[idx_g], rows_rgb, sem1)
            cp2 = pltpu.async_copy(flow_hbm.at[batch].at[idx_g], rows_flow, sem2)
            cp1.wait()
            cp2.wait()
            for h in range(2):
                acc = jnp.zeros((L,), jnp.float32)
                for j in range(3):
                    acc = acc + _sc_round_bf16(rows_flow[j, pl.ds(h * L, L)])
                km_v[pl.ds(h * L, L)] = acc / jnp.float32(3.0)
            pltpu.sync_copy(rows_rgb.at[pl.ds(0, 3)], krgb_hbm.at[batch])
            pltpu.sync_copy(km_v, km_hbm.at[batch])

    return sc1


def _make_sc2(b, m, f):
    mesh = plsc.VectorSubcoreMesh(core_axis_name="c", subcore_axis_name="s")

    @functools.partial(
        pl.kernel,
        mesh=mesh,
        out_type=jax.ShapeDtypeStruct((b, 5, f), jnp.float32),
        compiler_params=pltpu.CompilerParams(needs_layout_passes=False, use_tc_tiling_on_sc=False),
        scratch_types=[
            pltpu.VMEM((m,), jnp.float32),
            pltpu.VMEM((L, f), jnp.float32),
            pltpu.VMEM((5, f), jnp.float32),
            pltpu.SemaphoreType.DMA,
        ],
    )
    def sc2(s_hbm, suprgb_hbm, krgb_hbm, out_hbm, s_v, rows_v, out_v, sem):
        cid = lax.axis_index("c")
        sid = lax.axis_index("s")

        @pl.when(cid == 0)
        def _():
            batch = sid
            pltpu.sync_copy(s_hbm.at[batch], s_v)
            idx_g = _scan_topk(s_v, m, 2, largest=True)
            pltpu.async_copy(suprgb_hbm.at[batch].at[idx_g], rows_v, sem).wait()
            pltpu.sync_copy(krgb_hbm.at[batch], out_v.at[pl.ds(0, 3)])
            for j in range(2):
                for h in range(2):
                    out_v[3 + j, pl.ds(h * L, L)] = rows_v[j, pl.ds(h * L, L)]
            pltpu.sync_copy(out_v, out_hbm.at[batch])

    return sc2


def kernel(ref_rgb_feat, ref_flow_feat, sup_rgb_feat, sup_flow_feat):
    b, n, f = ref_flow_feat.shape
    m = sup_flow_feat.shape[1]

    s1 = _stage1_scores(ref_flow_feat)
    krgb, kmean = _make_sc1(b, n, f)(s1, ref_rgb_feat, ref_flow_feat)
    s2 = _stage2_scores(sup_flow_feat, kmean)
    out = _make_sc2(b, m, f)(s2, sup_rgb_feat, krgb)
    return out
